# Initial kernel scaffold; baseline (speedup 1.0000x reference)
import jax
import jax.numpy as jnp
from jax.experimental import pallas as pl


def kernel(x, edge_index, W1, a_src1, a_dst1, b1, W2, a_src2, a_dst2, b2):
    raise NotImplementedError("write your pallas kernel here")


# speedup vs baseline: 48.3162x; 48.3162x over previous
import functools

import jax
import jax.numpy as jnp
from jax import lax
from jax.experimental import pallas as pl
from jax.experimental.pallas import tpu as pltpu
from jax.experimental.pallas import tpu_sc as plsc

N = 10000
E = 320000
D = 128
HIM = 16
HEADS = 8
OUT = 64
NEG_SLOPE = 0.2

NPAD = 10016
NCORES = 2
NSUB = 16
NW = NCORES * NSUB
K = 128
EPAD = 331776
CPW = EPAD // (NW * K)
RPW = NPAD // NSUB
NEG = -1e30


def _seg_matrix(heads, ch):
    r = lax.broadcasted_iota(jnp.int32, (heads * ch, heads), 0) // ch
    c = lax.broadcasted_iota(jnp.int32, (heads * ch, heads), 1)
    return (r == c).astype(jnp.float32)


def _prep1_body(x_ref, w_ref, asrc_ref, adst_ref, aug_ref, dstt_ref, bvec_ref):
    h = jnp.dot(x_ref[...], w_ref[...], preferred_element_type=jnp.float32)
    seg = _seg_matrix(HEADS, HIM)
    asrc = jnp.dot(h * asrc_ref[...], seg, preferred_element_type=jnp.float32)
    adst = jnp.dot(h * adst_ref[...], seg, preferred_element_type=jnp.float32)
    bsum = (jnp.max(asrc, axis=0, keepdims=True)
            + jnp.max(adst, axis=0, keepdims=True))
    bvec_ref[...] = jnp.concatenate(
        [bsum, jnp.zeros((1, 8), jnp.float32)], axis=1)
    aug_ref[...] = jnp.concatenate(
        [h, asrc, jnp.full((N, 8), NEG, jnp.float32)], axis=1)
    dstt = jnp.concatenate([adst, jnp.full((N, 8), NEG, jnp.float32)], axis=1)
    dstt_ref[...] = jnp.concatenate(
        [dstt, jnp.full((NPAD - N, 16), NEG, jnp.float32)], axis=0)


_prep1 = pl.pallas_call(
    _prep1_body,
    out_shape=[
        jax.ShapeDtypeStruct((N, 144), jnp.float32),
        jax.ShapeDtypeStruct((NPAD, 16), jnp.float32),
        jax.ShapeDtypeStruct((1, 16), jnp.float32),
    ],
)


def _prep2_body(parts_ref, b1_ref, w2_ref, asrc2_ref, adst2_ref,
                aug_ref, dstt_ref, bvec_ref):
    aug1 = parts_ref[0] + parts_ref[1]
    msg = aug1[:N, 0:128]
    wsum = aug1[:N, 128:136]
    denf = jnp.dot(wsum, _seg_matrix(HEADS, HIM).T,
                   preferred_element_type=jnp.float32)
    x2 = jnp.maximum(msg / denf + b1_ref[...], 0.0)
    h2 = jnp.dot(x2, w2_ref[...], preferred_element_type=jnp.float32)

    lane0 = (lax.broadcasted_iota(jnp.int32, (1, 16), 1) == 0)
    a_s = asrc2_ref[...].T * lane0.astype(jnp.float32)
    a_d = adst2_ref[...].T * lane0.astype(jnp.float32)
    att_s = jnp.dot(h2, a_s, preferred_element_type=jnp.float32)
    att_d = jnp.dot(h2, a_d, preferred_element_type=jnp.float32)
    bsum = (jnp.max(att_s[:, 0:1], axis=0, keepdims=True)
            + jnp.max(att_d[:, 0:1], axis=0, keepdims=True))
    bvec_ref[...] = jnp.where(lane0, bsum, 0.0)
    att_s = jnp.where(lane0, att_s, NEG)
    att_d = jnp.where(lane0, att_d, NEG)
    aug_ref[...] = jnp.concatenate([h2, att_s], axis=1)
    dstt_ref[...] = jnp.concatenate(
        [att_d, jnp.full((NPAD - N, 16), NEG, jnp.float32)], axis=0)


_prep2 = pl.pallas_call(
    _prep2_body,
    out_shape=[
        jax.ShapeDtypeStruct((N, 80), jnp.float32),
        jax.ShapeDtypeStruct((NPAD, 16), jnp.float32),
        jax.ShapeDtypeStruct((1, 16), jnp.float32),
    ],
)


def _final_body(parts_ref, b2_ref, out_ref):
    aug2 = parts_ref[0] + parts_ref[1]
    msg = aug2[:N, 0:64]
    den = aug2[:N, 64:65]
    out_ref[...] = msg / den + b2_ref[...]


_final = pl.pallas_call(
    _final_body,
    out_shape=jax.ShapeDtypeStruct((N, OUT), jnp.float32),
)


def _edge_pass_body(row, hc, heads, ch,
                    aug_hbm, dstt_hbm, src_hbm, dst_hbm, bvec_hbm, out_hbm,
                    sidx, didx, rows, drows, bvec, acc, sem1, sem2):
    cid = lax.axis_index("c")
    sid = lax.axis_index("s")
    wid = sid * NCORES + cid

    def _zero_row(i, _):
        for j in range(row // 16):
            rows[i, pl.ds(j * 16, 16)] = jnp.zeros((16,), jnp.float32)
        return 0
    lax.fori_loop(0, K, _zero_row, 0)
    base_row = sid * RPW
    for t in range(RPW // K):
        pltpu.sync_copy(rows, acc.at[pl.ds(base_row + t * K, K)])
    rem = RPW % K
    if rem:
        pltpu.sync_copy(rows.at[pl.ds(0, rem)],
                        acc.at[pl.ds(base_row + (RPW // K) * K, rem)])
    plsc.subcore_barrier()

    pltpu.sync_copy(bvec_hbm, bvec)

    def _chunk(t, _):
        base = (wid * CPW + t) * K
        pltpu.sync_copy(src_hbm.at[pl.ds(base, K)], sidx)
        pltpu.sync_copy(dst_hbm.at[pl.ds(base, K)], didx)
        cp1 = pltpu.async_copy(aug_hbm.at[sidx], rows, sem1)
        cp2 = pltpu.async_copy(dstt_hbm.at[didx], drows, sem2)
        cp1.wait()
        cp2.wait()
        bv = bvec[...]

        def _edge(i, _):
            ev = rows[i, pl.ds(hc, 16)] + drows[i, :]
            ev = jnp.where(ev >= 0.0, ev, ev * NEG_SLOPE)
            wv = jnp.exp(ev - bv)
            rows[i, pl.ds(hc, 16)] = wv
            for j in range(heads):
                ws = wv[j]
                for v in range(ch // 16):
                    off = j * ch + v * 16
                    rows[i, pl.ds(off, 16)] = rows[i, pl.ds(off, 16)] * ws
            return 0
        lax.fori_loop(0, K, _edge, 0)
        pltpu.sync_copy(rows, acc.at[didx], add=True)
        return 0
    lax.fori_loop(0, CPW, _chunk, 0)

    plsc.subcore_barrier()
    pltpu.sync_copy(acc.at[pl.ds(base_row, RPW)],
                    out_hbm.at[cid].at[pl.ds(base_row, RPW)])


def _make_edge_pass(row, hc, heads, ch):
    return functools.partial(
        pl.kernel,
        out_type=jax.ShapeDtypeStruct((NCORES, NPAD, row), jnp.float32),
        mesh=plsc.VectorSubcoreMesh(core_axis_name="c", subcore_axis_name="s"),
        compiler_params=pltpu.CompilerParams(use_tc_tiling_on_sc=False),
        scratch_types=[
            pltpu.VMEM((K,), jnp.int32),
            pltpu.VMEM((K,), jnp.int32),
            pltpu.VMEM((K, row), jnp.float32),
            pltpu.VMEM((K, 16), jnp.float32),
            pltpu.VMEM((16,), jnp.float32),
            pltpu.VMEM_SHARED((NPAD, row), jnp.float32),
            pltpu.SemaphoreType.DMA,
            pltpu.SemaphoreType.DMA,
        ],
    )(functools.partial(_edge_pass_body, row, hc, heads, ch))


_edge_pass1 = _make_edge_pass(144, 128, HEADS, HIM)
_edge_pass2 = _make_edge_pass(80, 64, 1, OUT)


def kernel(x, edge_index, W1, a_src1, a_dst1, b1, W2, a_src2, a_dst2, b2):
    loops = jnp.arange(N, dtype=jnp.int32)
    npad_e = EPAD - (E + N)
    src = jnp.concatenate(
        [edge_index[0], loops, jnp.zeros((npad_e,), jnp.int32)])
    dst = jnp.concatenate(
        [edge_index[1], loops, jnp.full((npad_e,), N, jnp.int32)])

    aug1, dstt1, bvec1 = _prep1(
        x, W1, a_src1.reshape(1, HEADS * HIM), a_dst1.reshape(1, HEADS * HIM))
    parts1 = _edge_pass1(aug1, dstt1, src, dst, bvec1.reshape(16))

    aug2, dstt2, bvec2 = _prep2(
        parts1, b1.reshape(1, HEADS * HIM), W2, a_src2, a_dst2)
    parts2 = _edge_pass2(aug2, dstt2, src, dst, bvec2.reshape(16))

    return _final(parts2, b2.reshape(1, OUT))


# speedup vs baseline: 64.2306x; 1.3294x over previous
import functools

import jax
import jax.numpy as jnp
from jax import lax
from jax.experimental import pallas as pl
from jax.experimental.pallas import tpu as pltpu
from jax.experimental.pallas import tpu_sc as plsc

N = 10000
E = 320000
D = 128
HIM = 16
HEADS = 8
OUT = 64
NEG_SLOPE = 0.2

NPAD = 10016
NCORES = 2
NSUB = 16
NW = NCORES * NSUB
K = 96
EPAD = 331776
CPW = EPAD // (NW * K)
RPW = NPAD // NSUB
NEG = -1e30


def _seg_matrix(heads, ch):
    r = lax.broadcasted_iota(jnp.int32, (heads * ch, heads), 0) // ch
    c = lax.broadcasted_iota(jnp.int32, (heads * ch, heads), 1)
    return (r == c).astype(jnp.float32)


def _prep1_body(x_ref, w_ref, asrc_ref, adst_ref, aug_ref, dstt_ref, bvec_ref):
    h = jnp.dot(x_ref[...], w_ref[...], preferred_element_type=jnp.float32)
    seg = _seg_matrix(HEADS, HIM)
    asrc = jnp.dot(h * asrc_ref[...], seg, preferred_element_type=jnp.float32)
    adst = jnp.dot(h * adst_ref[...], seg, preferred_element_type=jnp.float32)
    bsum = (jnp.max(asrc, axis=0, keepdims=True)
            + jnp.max(adst, axis=0, keepdims=True))
    bvec_ref[...] = jnp.concatenate(
        [bsum, jnp.zeros((1, 8), jnp.float32)], axis=1)
    aug_ref[...] = jnp.concatenate(
        [h, asrc, jnp.full((N, 8), NEG, jnp.float32)], axis=1)
    dstt = jnp.concatenate([adst, jnp.full((N, 8), NEG, jnp.float32)], axis=1)
    dstt_ref[...] = jnp.concatenate(
        [dstt, jnp.full((NPAD - N, 16), NEG, jnp.float32)], axis=0)


_prep1 = pl.pallas_call(
    _prep1_body,
    out_shape=[
        jax.ShapeDtypeStruct((N, 144), jnp.float32),
        jax.ShapeDtypeStruct((NPAD, 16), jnp.float32),
        jax.ShapeDtypeStruct((1, 16), jnp.float32),
    ],
)


def _prep2_body(parts_ref, b1_ref, w2_ref, asrc2_ref, adst2_ref,
                aug_ref, dstt_ref, bvec_ref):
    aug1 = parts_ref[0] + parts_ref[1]
    msg = aug1[:N, 0:128]
    wsum = aug1[:N, 128:136]
    denf = jnp.dot(wsum, _seg_matrix(HEADS, HIM).T,
                   preferred_element_type=jnp.float32)
    x2 = jnp.maximum(msg / denf + b1_ref[...], 0.0)
    h2 = jnp.dot(x2, w2_ref[...], preferred_element_type=jnp.float32)

    lane0 = (lax.broadcasted_iota(jnp.int32, (1, 16), 1) == 0)
    a_s = asrc2_ref[...].T * lane0.astype(jnp.float32)
    a_d = adst2_ref[...].T * lane0.astype(jnp.float32)
    att_s = jnp.dot(h2, a_s, preferred_element_type=jnp.float32)
    att_d = jnp.dot(h2, a_d, preferred_element_type=jnp.float32)
    bsum = (jnp.max(att_s[:, 0:1], axis=0, keepdims=True)
            + jnp.max(att_d[:, 0:1], axis=0, keepdims=True))
    bvec_ref[...] = jnp.where(lane0, bsum, 0.0)
    att_s = jnp.where(lane0, att_s, NEG)
    att_d = jnp.where(lane0, att_d, NEG)
    aug_ref[...] = jnp.concatenate([h2, att_s], axis=1)
    dstt_ref[...] = jnp.concatenate(
        [att_d, jnp.full((NPAD - N, 16), NEG, jnp.float32)], axis=0)


_prep2 = pl.pallas_call(
    _prep2_body,
    out_shape=[
        jax.ShapeDtypeStruct((N, 80), jnp.float32),
        jax.ShapeDtypeStruct((NPAD, 16), jnp.float32),
        jax.ShapeDtypeStruct((1, 16), jnp.float32),
    ],
)


def _final_body(parts_ref, b2_ref, out_ref):
    aug2 = parts_ref[0] + parts_ref[1]
    msg = aug2[:N, 0:64]
    den = aug2[:N, 64:65]
    out_ref[...] = msg / den + b2_ref[...]


_final = pl.pallas_call(
    _final_body,
    out_shape=jax.ShapeDtypeStruct((N, OUT), jnp.float32),
)


def _edge_pass_body(row, hc, heads, ch,
                    aug_hbm, dstt_hbm, edges_hbm, bvec_hbm, out_hbm,
                    idx_a, idx_b, rows_a, rows_b, drows_a, drows_b,
                    bvec, acc, sem_a1, sem_a2, sem_b1, sem_b2,
                    sem_ia, sem_ib):
    cid = lax.axis_index("c")
    sid = lax.axis_index("s")
    wid = sid * NCORES + cid

    def _zero_row(i, _):
        for j in range(row // 16):
            rows_a[i, pl.ds(j * 16, 16)] = jnp.zeros((16,), jnp.float32)
        return 0
    lax.fori_loop(0, K, _zero_row, 0)
    base_row = sid * RPW
    for t in range(RPW // K):
        pltpu.sync_copy(rows_a, acc.at[pl.ds(base_row + t * K, K)])
    rem = RPW % K
    if rem:
        pltpu.sync_copy(rows_a.at[pl.ds(0, rem)],
                        acc.at[pl.ds(base_row + (RPW // K) * K, rem)])
    plsc.subcore_barrier()

    pltpu.sync_copy(bvec_hbm, bvec)
    base_c = wid * CPW

    def _issue_gather(idx, rows, drows, s1, s2):
        pltpu.async_copy(aug_hbm.at[idx.at[0]], rows, s1)
        pltpu.async_copy(dstt_hbm.at[idx.at[1]], drows, s2)

    def _wait_gather(rows, drows, s1, s2):
        pltpu.make_async_copy(aug_hbm.at[idx_a.at[0]], rows, s1).wait()
        pltpu.make_async_copy(dstt_hbm.at[idx_a.at[1]], drows, s2).wait()

    def _wait_idx(idx, sem):
        pltpu.make_async_copy(edges_hbm.at[0], idx, sem).wait()

    def _compute_scatter(idx, rows, drows):
        bv = bvec[...]

        def _edge(i, _):
            ev = rows[i, pl.ds(hc, 16)] + drows[i, :]
            ev = jnp.where(ev >= 0.0, ev, ev * NEG_SLOPE)
            wv = jnp.exp(ev - bv)
            rows[i, pl.ds(hc, 16)] = wv
            for j in range(heads):
                ws = wv[j]
                for v in range(ch // 16):
                    off = j * ch + v * 16
                    rows[i, pl.ds(off, 16)] = rows[i, pl.ds(off, 16)] * ws
            return 0
        lax.fori_loop(0, K, _edge, 0, unroll=2)
        pltpu.sync_copy(rows, acc.at[idx.at[1]], add=True)

    pltpu.sync_copy(edges_hbm.at[base_c], idx_a)
    pltpu.async_copy(edges_hbm.at[base_c + 1], idx_b, sem_ib)
    _issue_gather(idx_a, rows_a, drows_a, sem_a1, sem_a2)

    def _pair(r, _):
        c = base_c + r * 2
        _wait_gather(rows_a, drows_a, sem_a1, sem_a2)
        _wait_idx(idx_b, sem_ib)
        _issue_gather(idx_b, rows_b, drows_b, sem_b1, sem_b2)
        _compute_scatter(idx_a, rows_a, drows_a)
        pltpu.async_copy(edges_hbm.at[c + 2], idx_a, sem_ia)
        _wait_gather(rows_b, drows_b, sem_b1, sem_b2)
        _wait_idx(idx_a, sem_ia)
        _issue_gather(idx_a, rows_a, drows_a, sem_a1, sem_a2)
        _compute_scatter(idx_b, rows_b, drows_b)
        pltpu.async_copy(edges_hbm.at[c + 3], idx_b, sem_ib)
        return 0
    lax.fori_loop(0, CPW // 2, _pair, 0)
    _wait_gather(rows_a, drows_a, sem_a1, sem_a2)
    _wait_idx(idx_b, sem_ib)

    plsc.subcore_barrier()
    pltpu.sync_copy(acc.at[pl.ds(base_row, RPW)],
                    out_hbm.at[cid].at[pl.ds(base_row, RPW)])


def _make_edge_pass(row, hc, heads, ch):
    return functools.partial(
        pl.kernel,
        out_type=jax.ShapeDtypeStruct((NCORES, NPAD, row), jnp.float32),
        mesh=plsc.VectorSubcoreMesh(core_axis_name="c", subcore_axis_name="s"),
        compiler_params=pltpu.CompilerParams(use_tc_tiling_on_sc=False),
        scratch_types=[
            pltpu.VMEM((2, K), jnp.int32),
            pltpu.VMEM((2, K), jnp.int32),
            pltpu.VMEM((K, row), jnp.float32),
            pltpu.VMEM((K, row), jnp.float32),
            pltpu.VMEM((K, 16), jnp.float32),
            pltpu.VMEM((K, 16), jnp.float32),
            pltpu.VMEM((16,), jnp.float32),
            pltpu.VMEM_SHARED((NPAD, row), jnp.float32),
            pltpu.SemaphoreType.DMA,
            pltpu.SemaphoreType.DMA,
            pltpu.SemaphoreType.DMA,
            pltpu.SemaphoreType.DMA,
            pltpu.SemaphoreType.DMA,
            pltpu.SemaphoreType.DMA,
        ],
    )(functools.partial(_edge_pass_body, row, hc, heads, ch))


_edge_pass1 = _make_edge_pass(144, 128, HEADS, HIM)
_edge_pass2 = _make_edge_pass(80, 64, 1, OUT)


def kernel(x, edge_index, W1, a_src1, a_dst1, b1, W2, a_src2, a_dst2, b2):
    loops = jnp.arange(N, dtype=jnp.int32)
    npad_e = EPAD + 2 * K - (E + N)
    src = jnp.concatenate(
        [edge_index[0], loops,
         jnp.zeros((npad_e,), jnp.int32)]).reshape(-1, K)
    dst = jnp.concatenate(
        [edge_index[1], loops,
         jnp.full((npad_e,), N, jnp.int32)]).reshape(-1, K)
    edges = jnp.stack([src, dst], axis=1)

    aug1, dstt1, bvec1 = _prep1(
        x, W1, a_src1.reshape(1, HEADS * HIM), a_dst1.reshape(1, HEADS * HIM))
    parts1 = _edge_pass1(aug1, dstt1, edges, bvec1.reshape(16))

    aug2, dstt2, bvec2 = _prep2(
        parts1, b1.reshape(1, HEADS * HIM), W2, a_src2, a_dst2)
    parts2 = _edge_pass2(aug2, dstt2, edges, bvec2.reshape(16))

    return _final(parts2, b2.reshape(1, OUT))


# speedup vs baseline: 89.3210x; 1.3906x over previous
import functools

import jax
import jax.numpy as jnp
from jax import lax
from jax.experimental import pallas as pl
from jax.experimental.pallas import tpu as pltpu
from jax.experimental.pallas import tpu_sc as plsc

N = 10000
E = 320000
D = 128
HIM = 16
HEADS = 8
OUT = 64
NEG_SLOPE = 0.2

NPAD = 10016
NCORES = 2
NSUB = 16
NW = NCORES * NSUB
K = 96
EPAD = 331776
CPW = EPAD // (NW * K)
RPW = NPAD // NSUB
NEG = -1e30


def _seg_matrix(heads, ch):
    r = lax.broadcasted_iota(jnp.int32, (heads * ch, heads), 0) // ch
    c = lax.broadcasted_iota(jnp.int32, (heads * ch, heads), 1)
    return (r == c).astype(jnp.float32)


def _prep1_body(x_ref, w_ref, asrc_ref, adst_ref, aug_ref, dstt_ref, bvec_ref):
    h = jnp.dot(x_ref[...], w_ref[...], preferred_element_type=jnp.float32)
    seg = _seg_matrix(HEADS, HIM)
    asrc = jnp.dot(h * asrc_ref[...], seg, preferred_element_type=jnp.float32)
    adst = jnp.dot(h * adst_ref[...], seg, preferred_element_type=jnp.float32)
    bsum = (jnp.max(asrc, axis=0, keepdims=True)
            + jnp.max(adst, axis=0, keepdims=True))
    bvec_ref[...] = jnp.concatenate(
        [bsum, jnp.zeros((1, 8), jnp.float32)], axis=1)
    aug_ref[...] = jnp.concatenate(
        [h, asrc, jnp.full((N, 8), NEG, jnp.float32)], axis=1)
    dstt = jnp.concatenate([adst, jnp.full((N, 8), NEG, jnp.float32)], axis=1)
    dstt_ref[...] = jnp.concatenate(
        [dstt, jnp.full((NPAD - N, 16), NEG, jnp.float32)], axis=0)


_prep1 = pl.pallas_call(
    _prep1_body,
    out_shape=[
        jax.ShapeDtypeStruct((N, 144), jnp.float32),
        jax.ShapeDtypeStruct((NPAD, 16), jnp.float32),
        jax.ShapeDtypeStruct((1, 16), jnp.float32),
    ],
)


def _prep2_body(parts_ref, b1_ref, w2_ref, asrc2_ref, adst2_ref,
                aug_ref, dstt_ref, bvec_ref):
    aug1 = parts_ref[0] + parts_ref[1]
    msg = aug1[:N, 0:128]
    wsum = aug1[:N, 128:136]
    denf = jnp.dot(wsum, _seg_matrix(HEADS, HIM).T,
                   preferred_element_type=jnp.float32)
    x2 = jnp.maximum(msg / denf + b1_ref[...], 0.0)
    h2 = jnp.dot(x2, w2_ref[...], preferred_element_type=jnp.float32)

    lane0 = (lax.broadcasted_iota(jnp.int32, (1, 16), 1) == 0)
    a_s = asrc2_ref[...].T * lane0.astype(jnp.float32)
    a_d = adst2_ref[...].T * lane0.astype(jnp.float32)
    att_s = jnp.dot(h2, a_s, preferred_element_type=jnp.float32)
    att_d = jnp.dot(h2, a_d, preferred_element_type=jnp.float32)
    bsum = (jnp.max(att_s[:, 0:1], axis=0, keepdims=True)
            + jnp.max(att_d[:, 0:1], axis=0, keepdims=True))
    bvec_ref[...] = jnp.where(lane0, bsum, 0.0)
    att_s = jnp.where(lane0, att_s, NEG)
    att_d = jnp.where(lane0, att_d, NEG)
    aug_ref[...] = jnp.concatenate([h2, att_s], axis=1)
    dstt_ref[...] = jnp.concatenate(
        [att_d, jnp.full((NPAD - N, 16), NEG, jnp.float32)], axis=0)


_prep2 = pl.pallas_call(
    _prep2_body,
    out_shape=[
        jax.ShapeDtypeStruct((N, 80), jnp.float32),
        jax.ShapeDtypeStruct((NPAD, 16), jnp.float32),
        jax.ShapeDtypeStruct((1, 16), jnp.float32),
    ],
)


def _final_body(parts_ref, b2_ref, out_ref):
    aug2 = parts_ref[0] + parts_ref[1]
    msg = aug2[:N, 0:64]
    den = aug2[:N, 64:65]
    out_ref[...] = msg / den + b2_ref[...]


_final = pl.pallas_call(
    _final_body,
    out_shape=jax.ShapeDtypeStruct((N, OUT), jnp.float32),
)


def _edge_pass_body(row, hc, heads, ch,
                    aug_hbm, dstt_hbm, edges_hbm, bvec_hbm, out_hbm,
                    idx_a, idx_b, rows_a, rows_b, drows_a, drows_b,
                    bvec, acc, sem_a1, sem_a2, sem_b1, sem_b2,
                    sem_ia, sem_ib):
    cid = lax.axis_index("c")
    sid = lax.axis_index("s")
    wid = sid * NCORES + cid

    def _zero_row(i, _):
        for j in range(row // 16):
            rows_a[i, pl.ds(j * 16, 16)] = jnp.zeros((16,), jnp.float32)
        return 0
    lax.fori_loop(0, K, _zero_row, 0)
    base_row = sid * RPW
    for t in range(RPW // K):
        pltpu.sync_copy(rows_a, acc.at[pl.ds(base_row + t * K, K)])
    rem = RPW % K
    if rem:
        pltpu.sync_copy(rows_a.at[pl.ds(0, rem)],
                        acc.at[pl.ds(base_row + (RPW // K) * K, rem)])
    plsc.subcore_barrier()

    pltpu.sync_copy(bvec_hbm, bvec)
    base_c = wid * CPW

    def _issue_gather(idx, rows, drows, s1, s2):
        pltpu.async_copy(aug_hbm.at[idx.at[0]], rows, s1)
        pltpu.async_copy(dstt_hbm.at[idx.at[1]], drows, s2)

    def _wait_gather(rows, drows, s1, s2):
        pltpu.make_async_copy(aug_hbm.at[idx_a.at[0]], rows, s1).wait()
        pltpu.make_async_copy(dstt_hbm.at[idx_a.at[1]], drows, s2).wait()

    def _wait_idx(idx, sem):
        pltpu.make_async_copy(edges_hbm.at[0], idx, sem).wait()

    def _compute_scatter(idx, rows, drows):
        bv = bvec[...]

        @plsc.parallel_loop(0, K, 1, unroll=4)
        def _edge(i):
            ev = rows[i, pl.ds(hc, 16)] + drows[i, :]
            ev = jnp.where(ev >= 0.0, ev, ev * NEG_SLOPE)
            wv = jnp.exp(ev - bv)
            rows[i, pl.ds(hc, 16)] = wv
            for j in range(heads):
                ws = wv[j]
                for v in range(ch // 16):
                    off = j * ch + v * 16
                    rows[i, pl.ds(off, 16)] = rows[i, pl.ds(off, 16)] * ws
        pltpu.sync_copy(rows, acc.at[idx.at[1]], add=True)

    pltpu.sync_copy(edges_hbm.at[base_c], idx_a)
    pltpu.async_copy(edges_hbm.at[base_c + 1], idx_b, sem_ib)
    _issue_gather(idx_a, rows_a, drows_a, sem_a1, sem_a2)

    def _pair(r, _):
        c = base_c + r * 2
        _wait_gather(rows_a, drows_a, sem_a1, sem_a2)
        _wait_idx(idx_b, sem_ib)
        _issue_gather(idx_b, rows_b, drows_b, sem_b1, sem_b2)
        _compute_scatter(idx_a, rows_a, drows_a)
        pltpu.async_copy(edges_hbm.at[c + 2], idx_a, sem_ia)
        _wait_gather(rows_b, drows_b, sem_b1, sem_b2)
        _wait_idx(idx_a, sem_ia)
        _issue_gather(idx_a, rows_a, drows_a, sem_a1, sem_a2)
        _compute_scatter(idx_b, rows_b, drows_b)
        pltpu.async_copy(edges_hbm.at[c + 3], idx_b, sem_ib)
        return 0
    lax.fori_loop(0, CPW // 2, _pair, 0)
    _wait_gather(rows_a, drows_a, sem_a1, sem_a2)
    _wait_idx(idx_b, sem_ib)

    plsc.subcore_barrier()
    pltpu.sync_copy(acc.at[pl.ds(base_row, RPW)],
                    out_hbm.at[cid].at[pl.ds(base_row, RPW)])


def _make_edge_pass(row, hc, heads, ch):
    return functools.partial(
        pl.kernel,
        out_type=jax.ShapeDtypeStruct((NCORES, NPAD, row), jnp.float32),
        mesh=plsc.VectorSubcoreMesh(core_axis_name="c", subcore_axis_name="s"),
        compiler_params=pltpu.CompilerParams(use_tc_tiling_on_sc=False),
        scratch_types=[
            pltpu.VMEM((2, K), jnp.int32),
            pltpu.VMEM((2, K), jnp.int32),
            pltpu.VMEM((K, row), jnp.float32),
            pltpu.VMEM((K, row), jnp.float32),
            pltpu.VMEM((K, 16), jnp.float32),
            pltpu.VMEM((K, 16), jnp.float32),
            pltpu.VMEM((16,), jnp.float32),
            pltpu.VMEM_SHARED((NPAD, row), jnp.float32),
            pltpu.SemaphoreType.DMA,
            pltpu.SemaphoreType.DMA,
            pltpu.SemaphoreType.DMA,
            pltpu.SemaphoreType.DMA,
            pltpu.SemaphoreType.DMA,
            pltpu.SemaphoreType.DMA,
        ],
    )(functools.partial(_edge_pass_body, row, hc, heads, ch))


_edge_pass1 = _make_edge_pass(144, 128, HEADS, HIM)
_edge_pass2 = _make_edge_pass(80, 64, 1, OUT)


def kernel(x, edge_index, W1, a_src1, a_dst1, b1, W2, a_src2, a_dst2, b2):
    loops = jnp.arange(N, dtype=jnp.int32)
    npad_e = EPAD + 2 * K - (E + N)
    src = jnp.concatenate(
        [edge_index[0], loops,
         jnp.zeros((npad_e,), jnp.int32)]).reshape(-1, K)
    dst = jnp.concatenate(
        [edge_index[1], loops,
         jnp.full((npad_e,), N, jnp.int32)]).reshape(-1, K)
    edges = jnp.stack([src, dst], axis=1)

    aug1, dstt1, bvec1 = _prep1(
        x, W1, a_src1.reshape(1, HEADS * HIM), a_dst1.reshape(1, HEADS * HIM))
    parts1 = _edge_pass1(aug1, dstt1, edges, bvec1.reshape(16))

    aug2, dstt2, bvec2 = _prep2(
        parts1, b1.reshape(1, HEADS * HIM), W2, a_src2, a_dst2)
    parts2 = _edge_pass2(aug2, dstt2, edges, bvec2.reshape(16))

    return _final(parts2, b2.reshape(1, OUT))
